# baseline (speedup 1.0000x reference)
import functools

import jax
import jax.numpy as jnp
from jax import lax
from jax.experimental import pallas as pl
from jax.experimental.pallas import tpu as pltpu
from jax.experimental.pallas import tpu_sc as plsc

L = 16
NC, NS = 2, 16
NW = NC * NS
BATCH = 256
BB = BATCH // L
H, W = 513, 257
NPX = H * W
NPM = NPX - 1
PPW = NPM // NW
PK = 40
NCHUNK = PPW // PK
XROWS = 2 * NPX
NSH = 257
NSP = 384
RRW = NSP // NS
EPS = 1e-5


def _main_body(x_hbm, w_hbm, idx_hbm, out_hbm,
               x_buf, w_buf, idx_buf, acc, sem):
    cid = lax.axis_index("c")
    sid = lax.axis_index("s")
    wid = sid * NC + cid
    p0 = wid * PPW
    iota = lax.iota(jnp.int32, L)

    zeros = jnp.zeros((L,), jnp.float32)

    def zrow(i, c):
        for v in range(BATCH // L):
            acc[i, pl.ds(v * L, L)] = zeros
        return c

    lax.fori_loop(0, NSP, zrow, 0)

    def chunk_dmas(c, slot):
        pb = p0 + c * PK
        return (
            pltpu.make_async_copy(
                x_hbm.at[pl.ds(2 * pb, 2 * PK)], x_buf.at[slot], sem),
            pltpu.make_async_copy(
                w_hbm.at[pl.ds(pb, PK)], w_buf.at[slot], sem),
            pltpu.make_async_copy(
                idx_hbm.at[pl.ds(pb, PK)], idx_buf.at[slot], sem),
        )

    def start(c, slot):
        for d in chunk_dmas(c, slot):
            d.start()

    def wait(c, slot):
        for d in chunk_dmas(c, slot):
            d.wait()

    def compute(slot):
        @plsc.parallel_loop(0, PK)
        def pbody(pp):
            sv = jnp.full((L,), slot, jnp.int32)
            pv = jnp.full((L,), pp, jnp.int32)
            iv = plsc.load_gather(idx_buf, [sv, pv])
            wv = plsc.load_gather(w_buf, [sv, pv])
            for k in range(BB):
                dr, co = divmod(k * L, 128)
                xv = x_buf[slot, 2 * pp + dr, pl.ds(co, L)]
                yv = xv * xv * wv
                plsc.addupdate_scatter(acc, [iv, iota + (k * L)], yv)

    start(0, 0)

    def cbody(c, carry):
        slot = lax.rem(c, 2)
        wait(c, slot)

        @pl.when(c + 1 < NCHUNK)
        def _():
            start(c + 1, 1 - slot)

        compute(slot)
        return carry

    lax.fori_loop(0, NCHUNK, cbody, 0)

    pltpu.sync_copy(acc, out_hbm.at[wid])


def _reduce_body(slabs_hbm, out_hbm, acc8, tmp8):
    cid = lax.axis_index("c")
    sid = lax.axis_index("s")
    wid = sid * NC + cid

    def stripe(r0):
        pltpu.sync_copy(slabs_hbm.at[0, pl.ds(r0, 8)], acc8)

        def rbody(t, carry):
            pltpu.sync_copy(slabs_hbm.at[t, pl.ds(r0, 8)], tmp8)
            for r in range(8):
                for v in range(BATCH // L):
                    o = v * L
                    plsc.addupdate(acc8.at[r, pl.ds(o, L)],
                                   tmp8[r, pl.ds(o, L)])
            return carry

        lax.fori_loop(1, NW, rbody, 0)
        pltpu.sync_copy(acc8, out_hbm.at[pl.ds(r0, 8)])

    stripe(wid * 8)

    @pl.when(wid < (NSP // 8) - NW)
    def _():
        stripe((wid + NW) * 8)


@jax.jit
def _sc_spectrum(xp, wf, idxf):
    mesh = plsc.VectorSubcoreMesh(core_axis_name="c", subcore_axis_name="s")
    f = pl.kernel(
        _main_body,
        mesh=mesh,
        compiler_params=pltpu.CompilerParams(needs_layout_passes=False),
        out_type=jax.ShapeDtypeStruct((NW, NSP, BATCH), jnp.float32),
        scratch_types=[
            pltpu.VMEM((2, 2 * PK, 128), jnp.float32),
            pltpu.VMEM((2, PK), jnp.float32),
            pltpu.VMEM((2, PK), jnp.int32),
            pltpu.VMEM((NSP, BATCH), jnp.float32),
            pltpu.SemaphoreType.DMA,
        ],
    )
    red = pl.kernel(
        _reduce_body,
        mesh=mesh,
        compiler_params=pltpu.CompilerParams(needs_layout_passes=False),
        out_type=jax.ShapeDtypeStruct((NSP, BATCH), jnp.float32),
        scratch_types=[
            pltpu.VMEM((8, BATCH), jnp.float32),
            pltpu.VMEM((8, BATCH), jnp.float32),
        ],
    )
    return red(f(xp, wf, idxf))


def kernel(x, shells_weight, shell_index, shells_count):
    b, c, h, w_ = x.shape
    xp = jnp.transpose(x, (2, 3, 1, 0)).reshape(XROWS, 128)
    wf = shells_weight.reshape(-1)
    idxf = shell_index.reshape(-1)
    tot = _sc_spectrum(xp, wf, idxf)
    rec = 1.0 / (shells_count + EPS)
    out = (tot[:NSH] * rec[:, None]).T
    last = x[:, 0, h - 1, w_ - 1]
    contrib = (last * last) * wf[NPX - 1] / (shells_count[idxf[NPX - 1]] + EPS)
    out = out.at[:, idxf[NPX - 1]].add(contrib)
    return out.reshape(b, c, NSH)
